# baseline (speedup 1.0000x reference)
import functools

import jax
import jax.numpy as jnp
from jax import lax
from jax.experimental import pallas as pl
from jax.experimental.pallas import tpu as pltpu

_B = 2
_N = 200000
_C = 80
_K = 1000
_CHUNK = 2000
_NB = _N // _CHUNK

_OFF_STD = (0.1, 0.1, 0.2, 0.2)


_ROWS = 8
_BLK = _ROWS * _CHUNK
_NSTEPS = _B * _N // _BLK


def _score_body(pref, sref, cref):
    p = pref[...].reshape(_ROWS, _CHUNK, _C)
    m = jnp.max(p, axis=-1)
    iota = lax.broadcasted_iota(jnp.int32, (_ROWS, _CHUNK, _C), 2)
    cid = jnp.min(jnp.where(p == m[:, :, None], iota, _C), axis=-1)
    sref[...] = m
    cref[...] = cid.astype(jnp.int32)


def _scores_cids(probs):
    return pl.pallas_call(
        _score_body,
        grid=(_NSTEPS,),
        in_specs=[pl.BlockSpec((_BLK, _C), lambda i: (i, 0))],
        out_specs=[
            pl.BlockSpec((_ROWS, _CHUNK), lambda i: (i, 0)),
            pl.BlockSpec((_ROWS, _CHUNK), lambda i: (i, 0)),
        ],
        out_shape=[
            jax.ShapeDtypeStruct((_B * _N // _CHUNK, _CHUNK), jnp.float32),
            jax.ShapeDtypeStruct((_B * _N // _CHUNK, _CHUNK), jnp.int32),
        ],
    )(probs.reshape(_B * _N, _C))


def kernel(anchors, probs, offsets, window):
    scores3, cids3 = _scores_cids(probs)
    scores = scores3.reshape(_B, _N)
    class_ids = cids3.reshape(_B, _N)

    idx_list = []
    for b in range(_B):
        _, top_idx = lax.top_k(scores[b], _K)
        idx_list.append(
            jnp.stack([jnp.full((_K,), b, dtype=jnp.int32), top_idx.astype(jnp.int32)], axis=1)
        )
    anchor_idxes = jnp.concatenate(idx_list, axis=0)
    bi = anchor_idxes[:, 0]
    ai = anchor_idxes[:, 1]
    class_ids_sel = class_ids[bi, ai]
    scores_sel = scores[bi, ai]
    anchors_sel = anchors[bi, ai]
    off = offsets[bi, ai] * jnp.array(_OFF_STD, jnp.float32)
    h = anchors_sel[:, 2] - anchors_sel[:, 0]
    w = anchors_sel[:, 3] - anchors_sel[:, 1]
    cy = anchors_sel[:, 0] + 0.5 * h + off[:, 0] * h
    cx = anchors_sel[:, 1] + 0.5 * w + off[:, 1] * w
    h = h * jnp.exp(off[:, 2])
    w = w * jnp.exp(off[:, 3])
    y1 = cy - 0.5 * h
    x1 = cx - 0.5 * w
    y2 = y1 + h
    x2 = x1 + w
    boxes = jnp.stack([y1, x1, y2, x2], axis=1)
    wy1, wx1, wy2, wx2 = window[0, 0], window[0, 1], window[0, 2], window[0, 3]
    refined = jnp.stack(
        [
            jnp.clip(boxes[:, 0], wy1, wy2),
            jnp.clip(boxes[:, 1], wx1, wx2),
            jnp.clip(boxes[:, 2], wy1, wy2),
            jnp.clip(boxes[:, 3], wx1, wx2),
        ],
        axis=1,
    )
    return (anchor_idxes, refined, class_ids_sel, scores_sel)


# speedup vs baseline: 1.3304x; 1.3304x over previous
import functools

import jax
import jax.numpy as jnp
from jax import lax
from jax.experimental import pallas as pl
from jax.experimental.pallas import tpu as pltpu

_B = 2
_N = 200000
_C = 80
_K = 1000
_CHUNK = 2500
_NB = _N // _CHUNK

_OFF_STD = (0.1, 0.1, 0.2, 0.2)


_ROWS = 8
_BLK = _ROWS * _CHUNK
_NBLK = _N // _BLK


def _score_body(pref, sref, cref):
    p = pref[0].reshape(_ROWS, _CHUNK, _C)
    m = jnp.max(p, axis=-1)
    iota = lax.broadcasted_iota(jnp.int32, (_ROWS, _CHUNK, _C), 2)
    cid = jnp.min(jnp.where(p == m[:, :, None], iota, _C), axis=-1)
    sref[...] = m
    cref[...] = cid.astype(jnp.int32)


def _scores_cids(probs):
    return pl.pallas_call(
        _score_body,
        grid=(_B, _NBLK),
        in_specs=[pl.BlockSpec((1, _BLK, _C), lambda b, i: (b, i, 0))],
        out_specs=[
            pl.BlockSpec((_ROWS, _CHUNK), lambda b, i: (b * _NBLK + i, 0)),
            pl.BlockSpec((_ROWS, _CHUNK), lambda b, i: (b * _NBLK + i, 0)),
        ],
        out_shape=[
            jax.ShapeDtypeStruct((_B * _N // _CHUNK, _CHUNK), jnp.float32),
            jax.ShapeDtypeStruct((_B * _N // _CHUNK, _CHUNK), jnp.int32),
        ],
    )(probs)


def kernel(anchors, probs, offsets, window):
    scores3, cids3 = _scores_cids(probs)
    scores = scores3.reshape(_B, _N)
    class_ids = cids3.reshape(_B, _N)

    idx_list = []
    for b in range(_B):
        _, top_idx = lax.top_k(scores[b], _K)
        idx_list.append(
            jnp.stack([jnp.full((_K,), b, dtype=jnp.int32), top_idx.astype(jnp.int32)], axis=1)
        )
    anchor_idxes = jnp.concatenate(idx_list, axis=0)
    bi = anchor_idxes[:, 0]
    ai = anchor_idxes[:, 1]
    class_ids_sel = class_ids[bi, ai]
    scores_sel = scores[bi, ai]
    anchors_sel = anchors[bi, ai]
    off = offsets[bi, ai] * jnp.array(_OFF_STD, jnp.float32)
    h = anchors_sel[:, 2] - anchors_sel[:, 0]
    w = anchors_sel[:, 3] - anchors_sel[:, 1]
    cy = anchors_sel[:, 0] + 0.5 * h + off[:, 0] * h
    cx = anchors_sel[:, 1] + 0.5 * w + off[:, 1] * w
    h = h * jnp.exp(off[:, 2])
    w = w * jnp.exp(off[:, 3])
    y1 = cy - 0.5 * h
    x1 = cx - 0.5 * w
    y2 = y1 + h
    x2 = x1 + w
    boxes = jnp.stack([y1, x1, y2, x2], axis=1)
    wy1, wx1, wy2, wx2 = window[0, 0], window[0, 1], window[0, 2], window[0, 3]
    refined = jnp.stack(
        [
            jnp.clip(boxes[:, 0], wy1, wy2),
            jnp.clip(boxes[:, 1], wx1, wx2),
            jnp.clip(boxes[:, 2], wy1, wy2),
            jnp.clip(boxes[:, 3], wx1, wx2),
        ],
        axis=1,
    )
    return (anchor_idxes, refined, class_ids_sel, scores_sel)


# speedup vs baseline: 1.3328x; 1.0018x over previous
import functools

import jax
import jax.numpy as jnp
from jax import lax
from jax.experimental import pallas as pl
from jax.experimental.pallas import tpu as pltpu

_B = 2
_N = 200000
_C = 80
_K = 1000
_CHUNK = 2500

_OFF_STD = (0.1, 0.1, 0.2, 0.2)

_ROWS = 8
_BLK = _ROWS * _CHUNK
_NBLK = _N // _BLK


def _score_body(pref, sref, cref):
    p = pref[0].reshape(_ROWS, _CHUNK, _C)
    m = jnp.max(p, axis=-1)
    iota = lax.broadcasted_iota(jnp.int32, (_ROWS, _CHUNK, _C), 2)
    cid = jnp.min(jnp.where(p == m[:, :, None], iota, _C), axis=-1)
    sref[...] = m
    cref[...] = cid.astype(jnp.int32)


def _scores_cids(probs):
    return pl.pallas_call(
        _score_body,
        grid=(_B, _NBLK),
        in_specs=[pl.BlockSpec((1, _BLK, _C), lambda b, i: (b, i, 0))],
        out_specs=[
            pl.BlockSpec((_ROWS, _CHUNK), lambda b, i: (b * _NBLK + i, 0)),
            pl.BlockSpec((_ROWS, _CHUNK), lambda b, i: (b * _NBLK + i, 0)),
        ],
        out_shape=[
            jax.ShapeDtypeStruct((_B * _N // _CHUNK, _CHUNK), jnp.float32),
            jax.ShapeDtypeStruct((_B * _N // _CHUNK, _CHUNK), jnp.int32),
        ],
    )(probs)


def kernel(anchors, probs, offsets, window):
    scores3, cids3 = _scores_cids(probs)
    scores = scores3.reshape(_B, _N)
    class_ids = cids3.reshape(_B, _N)

    idx_list = []
    for b in range(_B):
        _, top_idx = lax.top_k(scores[b], _K)
        idx_list.append(
            jnp.stack([jnp.full((_K,), b, dtype=jnp.int32), top_idx.astype(jnp.int32)], axis=1)
        )
    anchor_idxes = jnp.concatenate(idx_list, axis=0)
    bi = anchor_idxes[:, 0]
    ai = anchor_idxes[:, 1]
    class_ids_sel = class_ids[bi, ai]
    scores_sel = scores[bi, ai]
    anchors_sel = anchors[bi, ai]
    off = offsets[bi, ai] * jnp.array(_OFF_STD, jnp.float32)
    h = anchors_sel[:, 2] - anchors_sel[:, 0]
    w = anchors_sel[:, 3] - anchors_sel[:, 1]
    cy = anchors_sel[:, 0] + 0.5 * h + off[:, 0] * h
    cx = anchors_sel[:, 1] + 0.5 * w + off[:, 1] * w
    h = h * jnp.exp(off[:, 2])
    w = w * jnp.exp(off[:, 3])
    y1 = cy - 0.5 * h
    x1 = cx - 0.5 * w
    y2 = y1 + h
    x2 = x1 + w
    boxes = jnp.stack([y1, x1, y2, x2], axis=1)
    wy1, wx1, wy2, wx2 = window[0, 0], window[0, 1], window[0, 2], window[0, 3]
    refined = jnp.stack(
        [
            jnp.clip(boxes[:, 0], wy1, wy2),
            jnp.clip(boxes[:, 1], wx1, wx2),
            jnp.clip(boxes[:, 2], wy1, wy2),
            jnp.clip(boxes[:, 3], wx1, wx2),
        ],
        axis=1,
    )
    return (anchor_idxes, refined, class_ids_sel, scores_sel)
